# Initial kernel scaffold; baseline (speedup 1.0000x reference)
import jax
import jax.numpy as jnp
from jax.experimental import pallas as pl


def kernel(atomic_numbers, distances, idx_i, idx_j, d3_rcov, d3_rcn, d3_rc6, d3_r2r4):
    raise NotImplementedError("write your pallas kernel here")


# speedup vs baseline: 63.2515x; 63.2515x over previous
import functools

import jax
import jax.numpy as jnp
from jax import lax
from jax.experimental import pallas as pl
from jax.experimental.pallas import tpu as pltpu
from jax.experimental.pallas import tpu_sc as plsc

N_NODES = 100000
N_EDGES = 1600000
N_ELEM = 95
N_REF = 5

NC = 2
NS = 16
L = 16
NW = NC * NS

NB = 3136
NP = NB * NW
SL = NP // NS

EC = 1024
NCHUNK = 49
EW = EC * NCHUNK
EP = EW * NW

D3_K1 = 16.0
D3_K2 = 4.0 / 3.0
D3_K3 = -4.0
D3_S6 = 1.0
D3_S8 = 0.9171
D3_A1 = 0.3385
D3_A2 = 2.883
SQRT3 = 3.0 ** 0.5

_MESH = plsc.VectorSubcoreMesh(core_axis_name="c", subcore_axis_name="s",
                               num_cores=NC, num_subcores=NS)


def _worker_id():
    return lax.axis_index("c") * NS + lax.axis_index("s")


def _zero_acc(zero_v, acc):
    sid = lax.axis_index("s")

    @pl.loop(0, SL // L)
    def _(i):
        zero_v[pl.ds(i * L, L)] = jnp.zeros((L,), jnp.float32)

    pltpu.sync_copy(zero_v, acc.at[pl.ds(sid * SL, SL)])
    plsc.subcore_barrier()


def _acc_to_out(acc, out_h):
    cid = lax.axis_index("c")
    sid = lax.axis_index("s")
    plsc.subcore_barrier()
    pltpu.sync_copy(acc.at[pl.ds(sid * SL, SL)],
                    out_h.at[pl.ds(cid * NP + sid * SL, SL)])


@functools.partial(
    pl.kernel,
    out_type=jax.ShapeDtypeStruct((NC * NP,), jnp.float32),
    mesh=_MESH,
    compiler_params=pltpu.CompilerParams(needs_layout_passes=False, use_tc_tiling_on_sc=False),
    scratch_types=[
        pltpu.VMEM((EC,), jnp.int32),
        pltpu.VMEM((EC,), jnp.int32),
        pltpu.VMEM((EC,), jnp.float32),
        pltpu.VMEM((EC,), jnp.int32),
        pltpu.VMEM((EC,), jnp.int32),
        pltpu.VMEM((EC,), jnp.float32),
        pltpu.VMEM((96,), jnp.float32),
        pltpu.VMEM((SL,), jnp.float32),
        pltpu.VMEM_SHARED((NP,), jnp.float32),
        pltpu.SemaphoreType.DMA,
    ],
)
def _cn_kernel(z_h, ii_h, jj_h, d_h, rcov_h, out_h,
               ii_v, jj_v, d_v, zi_v, zj_v, val_v, rcov_v, zero_v, acc, sem):
    wid = _worker_id()
    pltpu.sync_copy(rcov_h, rcov_v)
    _zero_acc(zero_v, acc)

    ebase = wid * EW

    @pl.loop(0, NCHUNK)
    def _chunk(c):
        base = ebase + c * EC
        pltpu.sync_copy(ii_h.at[pl.ds(base, EC)], ii_v)
        pltpu.sync_copy(jj_h.at[pl.ds(base, EC)], jj_v)
        pltpu.sync_copy(d_h.at[pl.ds(base, EC)], d_v)
        pltpu.async_copy(z_h.at[ii_v], zi_v, sem).wait()
        pltpu.async_copy(z_h.at[jj_v], zj_v, sem).wait()

        @pl.loop(0, EC // L)
        def _(i):
            s = pl.ds(i * L, L)
            ri = plsc.load_gather(rcov_v, [zi_v[s]])
            rj = plsc.load_gather(rcov_v, [zj_v[s]])
            rco = D3_K2 * (ri + rj)
            t = jnp.exp(-D3_K1 * (rco / d_v[s] - 1.0))
            val_v[s] = 1.0 / (1.0 + t)

        pltpu.sync_copy(val_v, acc.at[ii_v], add=True)

    _acc_to_out(acc, out_h)


@functools.partial(
    pl.kernel,
    out_type=jax.ShapeDtypeStruct((NP, 8), jnp.float32),
    mesh=_MESH,
    compiler_params=pltpu.CompilerParams(needs_layout_passes=False, use_tc_tiling_on_sc=False),
    scratch_types=[
        pltpu.VMEM((NB,), jnp.float32),
        pltpu.VMEM((NB,), jnp.float32),
        pltpu.VMEM((NB,), jnp.int32),
        pltpu.VMEM((NB, 8), jnp.float32),
        pltpu.VMEM((480,), jnp.float32),
        pltpu.VMEM((96,), jnp.float32),
        pltpu.VMEM((96,), jnp.float32),
    ],
)
def _gw_kernel(cn_h, z_h, rcn_h, q_h, sq_h, out_h,
               cn0_v, cn1_v, z_v, feat_v, rcn_v, q_v, sq_v):
    wid = _worker_id()
    nb = wid * NB
    pltpu.sync_copy(rcn_h, rcn_v)
    pltpu.sync_copy(q_h, q_v)
    pltpu.sync_copy(sq_h, sq_v)
    pltpu.sync_copy(cn_h.at[pl.ds(nb, NB)], cn0_v)
    pltpu.sync_copy(cn_h.at[pl.ds(NP + nb, NB)], cn1_v)
    pltpu.sync_copy(z_h.at[pl.ds(nb, NB)], z_v)

    iota = lax.iota(jnp.int32, L)
    cols = [jnp.full((L,), k, jnp.int32) for k in range(8)]

    @pl.loop(0, NB // L)
    def _(i):
        s = pl.ds(i * L, L)
        z = z_v[s]
        cn = cn0_v[s] + cn1_v[s]
        zb = z * N_REF
        r = [plsc.load_gather(rcn_v, [zb + k]) for k in range(N_REF)]
        maxcn = r[0]
        for k in range(1, N_REF):
            maxcn = jnp.maximum(maxcn, r[k])
        w = []
        norm = None
        for k in range(N_REF):
            d = cn - r[k]
            wk = jnp.exp(D3_K3 * d * d)
            w.append(wk)
            norm = wk if norm is None else norm + wk
        exc = norm < 1e-30
        safe = jnp.where(exc, 1.0, norm)
        rows = i * L + iota
        for k in range(N_REF):
            gwk = jnp.where(exc, jnp.where(r[k] == maxcn, 1.0, 0.0),
                            w[k] / safe)
            plsc.store_scatter(feat_v, [rows, cols[k]], gwk)
        plsc.store_scatter(feat_v, [rows, cols[5]],
                           plsc.load_gather(q_v, [z]))
        plsc.store_scatter(feat_v, [rows, cols[6]],
                           plsc.load_gather(sq_v, [z]))
        plsc.store_scatter(feat_v, [rows, cols[7]], z.astype(jnp.float32))

    pltpu.sync_copy(feat_v, out_h.at[pl.ds(nb, NB), :])


@functools.partial(
    pl.kernel,
    out_type=jax.ShapeDtypeStruct((NC * NP,), jnp.float32),
    mesh=_MESH,
    compiler_params=pltpu.CompilerParams(needs_layout_passes=False, use_tc_tiling_on_sc=False),
    scratch_types=[
        pltpu.VMEM((EC,), jnp.int32),
        pltpu.VMEM((EC,), jnp.int32),
        pltpu.VMEM((EC,), jnp.float32),
        pltpu.VMEM((EC,), jnp.int32),
        pltpu.VMEM((EC, 8), jnp.float32),
        pltpu.VMEM((EC, 8), jnp.float32),
        pltpu.VMEM((EC, 32), jnp.float32),
        pltpu.VMEM((EC,), jnp.float32),
        pltpu.VMEM((SL,), jnp.float32),
        pltpu.VMEM_SHARED((NP,), jnp.float32),
        pltpu.SemaphoreType.DMA,
    ],
)
def _edisp_kernel(ii_h, jj_h, d_h, feat_h, rc6_h, out_h,
                  ii_v, jj_v, d_v, p_v, wi_v, wj_v, rows_v, val_v,
                  zero_v, acc, sem):
    wid = _worker_id()
    _zero_acc(zero_v, acc)

    ebase = wid * EW
    iota = lax.iota(jnp.int32, L)
    cols = [jnp.full((L,), k, jnp.int32) for k in range(32)]

    @pl.loop(0, NCHUNK)
    def _chunk(c):
        base = ebase + c * EC
        pltpu.sync_copy(ii_h.at[pl.ds(base, EC)], ii_v)
        pltpu.sync_copy(jj_h.at[pl.ds(base, EC)], jj_v)
        pltpu.sync_copy(d_h.at[pl.ds(base, EC)], d_v)
        pltpu.async_copy(feat_h.at[ii_v], wi_v, sem).wait()
        pltpu.async_copy(feat_h.at[jj_v], wj_v, sem).wait()

        @pl.loop(0, EC // L)
        def _(i):
            rows = i * L + iota
            zi = plsc.load_gather(wi_v, [rows, cols[7]]).astype(jnp.int32)
            zj = plsc.load_gather(wj_v, [rows, cols[7]]).astype(jnp.int32)
            p_v[pl.ds(i * L, L)] = zi * N_ELEM + zj

        pltpu.async_copy(rc6_h.at[p_v], rows_v, sem).wait()

        @pl.loop(0, EC // L)
        def _(i):
            s = pl.ds(i * L, L)
            rows = i * L + iota
            wi = [plsc.load_gather(wi_v, [rows, cols[a]])
                  for a in range(N_REF)]
            wj = [plsc.load_gather(wj_v, [rows, cols[b]])
                  for b in range(N_REF)]
            c6 = None
            for a in range(N_REF):
                ta = None
                for b in range(N_REF):
                    rab = plsc.load_gather(rows_v, [rows, cols[a * N_REF + b]])
                    t = wj[b] * rab
                    ta = t if ta is None else ta + t
                t = wi[a] * ta
                c6 = t if c6 is None else c6 + t
            qi = plsc.load_gather(wi_v, [rows, cols[5]])
            qj = plsc.load_gather(wj_v, [rows, cols[5]])
            sqi = plsc.load_gather(wi_v, [rows, cols[6]])
            sqj = plsc.load_gather(wj_v, [rows, cols[6]])
            qq = 3.0 * qi * qj
            c8 = c6 * qq
            rr = D3_A1 * SQRT3 * sqi * sqj + D3_A2
            r = d_v[s]
            r2 = r * r
            r6 = r2 * r2 * r2
            r8 = r6 * r2
            rr2 = rr * rr
            rr6 = rr2 * rr2 * rr2
            rr8 = rr6 * rr2
            val_v[s] = -0.5 * (D3_S6 * c6 / (r6 + rr6)
                               + D3_S8 * c8 / (r8 + rr8))

        pltpu.sync_copy(val_v, acc.at[ii_v], add=True)

    _acc_to_out(acc, out_h)


def _add_body(x_ref, o_ref):
    o_ref[...] = x_ref[0] + x_ref[1]


_add_call = pl.pallas_call(
    _add_body,
    out_shape=jax.ShapeDtypeStruct((NP // 128, 128), jnp.float32),
)


def kernel(atomic_numbers, distances, idx_i, idx_j,
           d3_rcov, d3_rcn, d3_rc6, d3_r2r4):
    z = atomic_numbers.astype(jnp.int32)
    ii = idx_i.astype(jnp.int32)
    jj = idx_j.astype(jnp.int32)
    dist = distances.astype(jnp.float32)

    zp = jnp.pad(z, (0, NP - N_NODES))
    pad_e = EP - N_EDGES
    iip = jnp.pad(ii, (0, pad_e), constant_values=N_NODES)
    jjp = jnp.pad(jj, (0, pad_e))
    dp = jnp.pad(dist, (0, pad_e), constant_values=1.0)

    rcov96 = jnp.pad(d3_rcov.astype(jnp.float32), (0, 96 - N_ELEM))
    rcn480 = jnp.pad(d3_rcn.astype(jnp.float32).reshape(-1),
                     (0, 480 - N_ELEM * N_REF))
    q96 = jnp.pad(d3_r2r4.astype(jnp.float32), (0, 96 - N_ELEM))
    sq96 = jnp.sqrt(q96)
    rc6p = jnp.pad(
        d3_rc6.astype(jnp.float32).reshape(N_ELEM * N_ELEM, N_REF * N_REF),
        ((0, 0), (0, 32 - N_REF * N_REF)))

    cn_parts = _cn_kernel(zp, iip, jjp, dp, rcov96)
    feat = _gw_kernel(cn_parts, zp, rcn480, q96, sq96)
    e_parts = _edisp_kernel(iip, jjp, dp, feat, rc6p)
    edisp = _add_call(e_parts.reshape(NC, NP // 128, 128))
    return edisp.reshape(NP)[:N_NODES]


# speedup vs baseline: 70.1977x; 1.1098x over previous
import functools

import jax
import jax.numpy as jnp
from jax import lax
from jax.experimental import pallas as pl
from jax.experimental.pallas import tpu as pltpu
from jax.experimental.pallas import tpu_sc as plsc

N_NODES = 100000
N_EDGES = 1600000
N_ELEM = 95
N_REF = 5

NC = 2
NS = 16
L = 16
NW = NC * NS

NB = 3136
NP = NB * NW
SL = NP // NS

EC = 1024
NCHUNK = 50
EW = EC * NCHUNK
EP = EW * NW

D3_K1 = 16.0
D3_K2 = 4.0 / 3.0
D3_K3 = -4.0
D3_S6 = 1.0
D3_S8 = 0.9171
D3_A1 = 0.3385
D3_A2 = 2.883
SQRT3 = 3.0 ** 0.5

_MESH = plsc.VectorSubcoreMesh(core_axis_name="c", subcore_axis_name="s",
                               num_cores=NC, num_subcores=NS)
_PARAMS = pltpu.CompilerParams(needs_layout_passes=False,
                               use_tc_tiling_on_sc=False)


def _worker_id():
    return lax.axis_index("c") * NS + lax.axis_index("s")


def _zero_acc(zero_v, acc):
    sid = lax.axis_index("s")

    @pl.loop(0, SL // L)
    def _(i):
        zero_v[pl.ds(i * L, L)] = jnp.zeros((L,), jnp.float32)

    pltpu.sync_copy(zero_v, acc.at[pl.ds(sid * SL, SL)])
    plsc.subcore_barrier()


def _acc_to_out(acc, out_h):
    cid = lax.axis_index("c")
    sid = lax.axis_index("s")
    plsc.subcore_barrier()
    pltpu.sync_copy(acc.at[pl.ds(sid * SL, SL)],
                    out_h.at[pl.ds(cid * NP + sid * SL, SL)])


@functools.partial(
    pl.kernel,
    out_type=[jax.ShapeDtypeStruct((NC * NP,), jnp.float32),
              jax.ShapeDtypeStruct((EP,), jnp.int32)],
    mesh=_MESH,
    compiler_params=_PARAMS,
    scratch_types=[
        [pltpu.VMEM((EC,), jnp.int32)] * 2,
        [pltpu.VMEM((EC,), jnp.int32)] * 2,
        [pltpu.VMEM((EC,), jnp.float32)] * 2,
        [pltpu.VMEM((EC,), jnp.int32)] * 2,
        [pltpu.VMEM((EC,), jnp.int32)] * 2,
        [pltpu.VMEM((EC,), jnp.int32)] * 2,
        pltpu.VMEM((EC,), jnp.float32),
        pltpu.VMEM((96,), jnp.float32),
        pltpu.VMEM((SL,), jnp.float32),
        pltpu.VMEM_SHARED((NP,), jnp.float32),
        [pltpu.SemaphoreType.DMA] * 2,
        [pltpu.SemaphoreType.DMA] * 2,
        [pltpu.SemaphoreType.DMA] * 2,
    ],
)
def _cn_kernel(z_h, ii_h, jj_h, d_h, rcov_h, cn_out, p_out,
               ii_v, jj_v, d_v, zi_v, zj_v, p_v, val_v, rcov_v, zero_v, acc,
               seml, semg, semp):
    wid = _worker_id()
    pltpu.sync_copy(rcov_h, rcov_v)
    _zero_acc(zero_v, acc)

    ebase = wid * EW

    def base(c):
        return ebase + c * EC

    def issue_lin(c, b):
        pltpu.async_copy(ii_h.at[pl.ds(base(c), EC)], ii_v[b], seml[b])
        pltpu.async_copy(jj_h.at[pl.ds(base(c), EC)], jj_v[b], seml[b])
        pltpu.async_copy(d_h.at[pl.ds(base(c), EC)], d_v[b], seml[b])

    def wait_lin(b):
        pltpu.make_async_copy(ii_h.at[pl.ds(0, EC)], ii_v[b], seml[b]).wait()
        pltpu.make_async_copy(jj_h.at[pl.ds(0, EC)], jj_v[b], seml[b]).wait()
        pltpu.make_async_copy(d_h.at[pl.ds(0, EC)], d_v[b], seml[b]).wait()

    def issue_gat(b):
        pltpu.async_copy(z_h.at[ii_v[b]], zi_v[b], semg[b])
        pltpu.async_copy(z_h.at[jj_v[b]], zj_v[b], semg[b])

    def wait_gat(b):
        pltpu.make_async_copy(z_h.at[pl.ds(0, EC)], zi_v[b], semg[b]).wait()
        pltpu.make_async_copy(z_h.at[pl.ds(0, EC)], zj_v[b], semg[b]).wait()

    def wait_pwrite(b):
        pltpu.make_async_copy(p_v[b], p_out.at[pl.ds(0, EC)], semp[b]).wait()

    issue_lin(0, 0)
    wait_lin(0)
    issue_gat(0)
    issue_lin(1, 1)

    @pl.loop(0, NCHUNK // 2)
    def _pair(m):
        for b in (0, 1):
            c = 2 * m + b
            o = 1 - b
            if b == 0:
                wait_lin(o)
                issue_gat(o)
            else:
                @pl.when(m < NCHUNK // 2 - 1)
                def _():
                    wait_lin(o)
                    issue_gat(o)
            wait_gat(b)

            @pl.when(m >= 1)
            def _():
                wait_pwrite(b)

            @pl.loop(0, EC // L)
            def _(i):
                s = pl.ds(i * L, L)
                zi = zi_v[b][s]
                zj = zj_v[b][s]
                ri = plsc.load_gather(rcov_v, [zi])
                rj = plsc.load_gather(rcov_v, [zj])
                rco = D3_K2 * (ri + rj)
                t = jnp.exp(-D3_K1 * (rco / d_v[b][s] - 1.0))
                val_v[s] = 1.0 / (1.0 + t)
                p_v[b][s] = zi * N_ELEM + zj

            pltpu.async_copy(p_v[b], p_out.at[pl.ds(base(c), EC)], semp[b])
            pltpu.sync_copy(val_v, acc.at[ii_v[b]], add=True)

            @pl.when(m < NCHUNK // 2 - 1)
            def _():
                issue_lin(c + 2, b)

    wait_pwrite(0)
    wait_pwrite(1)
    _acc_to_out(acc, cn_out)


@functools.partial(
    pl.kernel,
    out_type=jax.ShapeDtypeStruct((NP, 8), jnp.float32),
    mesh=_MESH,
    compiler_params=_PARAMS,
    scratch_types=[
        pltpu.VMEM((NB,), jnp.float32),
        pltpu.VMEM((NB,), jnp.float32),
        pltpu.VMEM((NB,), jnp.int32),
        pltpu.VMEM((NB, 8), jnp.float32),
        pltpu.VMEM((480,), jnp.float32),
        pltpu.VMEM((96,), jnp.float32),
        pltpu.VMEM((96,), jnp.float32),
    ],
)
def _gw_kernel(cn_h, z_h, rcn_h, q_h, sq_h, out_h,
               cn0_v, cn1_v, z_v, feat_v, rcn_v, q_v, sq_v):
    wid = _worker_id()
    nb = wid * NB
    pltpu.sync_copy(rcn_h, rcn_v)
    pltpu.sync_copy(q_h, q_v)
    pltpu.sync_copy(sq_h, sq_v)
    pltpu.sync_copy(cn_h.at[pl.ds(nb, NB)], cn0_v)
    pltpu.sync_copy(cn_h.at[pl.ds(NP + nb, NB)], cn1_v)
    pltpu.sync_copy(z_h.at[pl.ds(nb, NB)], z_v)

    iota = lax.iota(jnp.int32, L)
    cols = [jnp.full((L,), k, jnp.int32) for k in range(8)]

    @pl.loop(0, NB // L)
    def _(i):
        s = pl.ds(i * L, L)
        z = z_v[s]
        cn = cn0_v[s] + cn1_v[s]
        zb = z * N_REF
        r = [plsc.load_gather(rcn_v, [zb + k]) for k in range(N_REF)]
        maxcn = r[0]
        for k in range(1, N_REF):
            maxcn = jnp.maximum(maxcn, r[k])
        w = []
        norm = None
        for k in range(N_REF):
            d = cn - r[k]
            wk = jnp.exp(D3_K3 * d * d)
            w.append(wk)
            norm = wk if norm is None else norm + wk
        exc = norm < 1e-30
        safe = jnp.where(exc, 1.0, norm)
        rows = i * L + iota
        for k in range(N_REF):
            gwk = jnp.where(exc, jnp.where(r[k] == maxcn, 1.0, 0.0),
                            w[k] / safe)
            plsc.store_scatter(feat_v, [rows, cols[k]], gwk)
        plsc.store_scatter(feat_v, [rows, cols[5]],
                           plsc.load_gather(q_v, [z]))
        plsc.store_scatter(feat_v, [rows, cols[6]],
                           plsc.load_gather(sq_v, [z]))
        plsc.store_scatter(feat_v, [rows, cols[7]], z.astype(jnp.float32))

    pltpu.sync_copy(feat_v, out_h.at[pl.ds(nb, NB), :])


@functools.partial(
    pl.kernel,
    out_type=jax.ShapeDtypeStruct((NC * NP,), jnp.float32),
    mesh=_MESH,
    compiler_params=_PARAMS,
    scratch_types=[
        [pltpu.VMEM((EC,), jnp.int32)] * 2,
        [pltpu.VMEM((EC,), jnp.int32)] * 2,
        [pltpu.VMEM((EC,), jnp.int32)] * 2,
        [pltpu.VMEM((EC,), jnp.float32)] * 2,
        [pltpu.VMEM((EC, 8), jnp.float32)] * 2,
        [pltpu.VMEM((EC, 8), jnp.float32)] * 2,
        [pltpu.VMEM((EC, 32), jnp.float32)] * 2,
        pltpu.VMEM((EC,), jnp.float32),
        pltpu.VMEM((SL,), jnp.float32),
        pltpu.VMEM_SHARED((NP,), jnp.float32),
        [pltpu.SemaphoreType.DMA] * 2,
        [pltpu.SemaphoreType.DMA] * 2,
    ],
)
def _edisp_kernel(ii_h, jj_h, d_h, p_h, feat_h, rc6_h, out_h,
                  ii_v, jj_v, p_v, d_v, wi_v, wj_v, rows_v, val_v, zero_v,
                  acc, seml, semg):
    wid = _worker_id()
    _zero_acc(zero_v, acc)

    ebase = wid * EW
    iota = lax.iota(jnp.int32, L)
    cols = [jnp.full((L,), k, jnp.int32) for k in range(32)]

    def base(c):
        return ebase + c * EC

    def issue_lin(c, b):
        pltpu.async_copy(ii_h.at[pl.ds(base(c), EC)], ii_v[b], seml[b])
        pltpu.async_copy(jj_h.at[pl.ds(base(c), EC)], jj_v[b], seml[b])
        pltpu.async_copy(p_h.at[pl.ds(base(c), EC)], p_v[b], seml[b])
        pltpu.async_copy(d_h.at[pl.ds(base(c), EC)], d_v[b], seml[b])

    def wait_lin(b):
        pltpu.make_async_copy(ii_h.at[pl.ds(0, EC)], ii_v[b], seml[b]).wait()
        pltpu.make_async_copy(jj_h.at[pl.ds(0, EC)], jj_v[b], seml[b]).wait()
        pltpu.make_async_copy(p_h.at[pl.ds(0, EC)], p_v[b], seml[b]).wait()
        pltpu.make_async_copy(d_h.at[pl.ds(0, EC)], d_v[b], seml[b]).wait()

    def issue_gat(b):
        pltpu.async_copy(feat_h.at[ii_v[b]], wi_v[b], semg[b])
        pltpu.async_copy(feat_h.at[jj_v[b]], wj_v[b], semg[b])
        pltpu.async_copy(rc6_h.at[p_v[b]], rows_v[b], semg[b])

    def wait_gat(b):
        pltpu.make_async_copy(feat_h.at[pl.ds(0, EC), :], wi_v[b],
                              semg[b]).wait()
        pltpu.make_async_copy(feat_h.at[pl.ds(0, EC), :], wj_v[b],
                              semg[b]).wait()
        pltpu.make_async_copy(rc6_h.at[pl.ds(0, EC), :], rows_v[b],
                              semg[b]).wait()

    issue_lin(0, 0)
    wait_lin(0)
    issue_gat(0)
    issue_lin(1, 1)

    @pl.loop(0, NCHUNK // 2)
    def _pair(m):
      for b in (0, 1):
        c = 2 * m + b
        o = 1 - b
        if b == 0:
            wait_lin(o)
            issue_gat(o)
        else:
            @pl.when(m < NCHUNK // 2 - 1)
            def _():
                wait_lin(o)
                issue_gat(o)
        wait_gat(b)

        @pl.loop(0, EC // L)
        def _(i):
            s = pl.ds(i * L, L)
            rows = i * L + iota
            wi = [plsc.load_gather(wi_v[b], [rows, cols[a]])
                  for a in range(N_REF)]
            wj = [plsc.load_gather(wj_v[b], [rows, cols[k]])
                  for k in range(N_REF)]
            c6 = None
            for a in range(N_REF):
                ta = None
                for k in range(N_REF):
                    rab = plsc.load_gather(rows_v[b],
                                           [rows, cols[a * N_REF + k]])
                    t = wj[k] * rab
                    ta = t if ta is None else ta + t
                t = wi[a] * ta
                c6 = t if c6 is None else c6 + t
            qi = plsc.load_gather(wi_v[b], [rows, cols[5]])
            qj = plsc.load_gather(wj_v[b], [rows, cols[5]])
            sqi = plsc.load_gather(wi_v[b], [rows, cols[6]])
            sqj = plsc.load_gather(wj_v[b], [rows, cols[6]])
            qq = 3.0 * qi * qj
            c8 = c6 * qq
            rr = D3_A1 * SQRT3 * sqi * sqj + D3_A2
            r = d_v[b][s]
            r2 = r * r
            r6 = r2 * r2 * r2
            r8 = r6 * r2
            rr2 = rr * rr
            rr6 = rr2 * rr2 * rr2
            rr8 = rr6 * rr2
            val_v[s] = -0.5 * (D3_S6 * c6 / (r6 + rr6)
                               + D3_S8 * c8 / (r8 + rr8))

        pltpu.sync_copy(val_v, acc.at[ii_v[b]], add=True)

        @pl.when(m < NCHUNK // 2 - 1)
        def _():
            issue_lin(c + 2, b)

    _acc_to_out(acc, out_h)


def _add_body(x_ref, o_ref):
    o_ref[...] = x_ref[0] + x_ref[1]


_add_call = pl.pallas_call(
    _add_body,
    out_shape=jax.ShapeDtypeStruct((NP // 128, 128), jnp.float32),
)


def kernel(atomic_numbers, distances, idx_i, idx_j,
           d3_rcov, d3_rcn, d3_rc6, d3_r2r4):
    z = atomic_numbers.astype(jnp.int32)
    ii = idx_i.astype(jnp.int32)
    jj = idx_j.astype(jnp.int32)
    dist = distances.astype(jnp.float32)

    zp = jnp.pad(z, (0, NP - N_NODES))
    pad_e = EP - N_EDGES
    iip = jnp.pad(ii, (0, pad_e), constant_values=N_NODES)
    jjp = jnp.pad(jj, (0, pad_e))
    dp = jnp.pad(dist, (0, pad_e), constant_values=1.0)

    rcov96 = jnp.pad(d3_rcov.astype(jnp.float32), (0, 96 - N_ELEM))
    rcn480 = jnp.pad(d3_rcn.astype(jnp.float32).reshape(-1),
                     (0, 480 - N_ELEM * N_REF))
    q96 = jnp.pad(d3_r2r4.astype(jnp.float32), (0, 96 - N_ELEM))
    sq96 = jnp.sqrt(q96)
    rc6p = jnp.pad(
        d3_rc6.astype(jnp.float32).reshape(N_ELEM * N_ELEM, N_REF * N_REF),
        ((0, 0), (0, 32 - N_REF * N_REF)))

    cn_parts, pidx = _cn_kernel(zp, iip, jjp, dp, rcov96)
    feat = _gw_kernel(cn_parts, zp, rcn480, q96, sq96)
    e_parts = _edisp_kernel(iip, jjp, dp, pidx, feat, rc6p)
    edisp = _add_call(e_parts.reshape(NC, NP // 128, 128))
    return edisp.reshape(NP)[:N_NODES]


# speedup vs baseline: 71.1965x; 1.0142x over previous
import functools

import jax
import jax.numpy as jnp
from jax import lax
from jax.experimental import pallas as pl
from jax.experimental.pallas import tpu as pltpu
from jax.experimental.pallas import tpu_sc as plsc

N_NODES = 100000
N_EDGES = 1600000
N_ELEM = 95
N_REF = 5

NC = 2
NS = 16
L = 16
NW = NC * NS

NB = 3136
NP = NB * NW
SL = NP // NS

EC = 1024
NCHUNK = 50
EW = EC * NCHUNK
EP = EW * NW

D3_K1 = 16.0
D3_K2 = 4.0 / 3.0
D3_K3 = -4.0
D3_S6 = 1.0
D3_S8 = 0.9171
D3_A1 = 0.3385
D3_A2 = 2.883
SQRT3 = 3.0 ** 0.5

_MESH = plsc.VectorSubcoreMesh(core_axis_name="c", subcore_axis_name="s",
                               num_cores=NC, num_subcores=NS)
_PARAMS = pltpu.CompilerParams(needs_layout_passes=False,
                               use_tc_tiling_on_sc=False)


def _worker_id():
    return lax.axis_index("c") * NS + lax.axis_index("s")


def _zero_acc(zero_v, acc):
    sid = lax.axis_index("s")

    @pl.loop(0, SL // L)
    def _(i):
        zero_v[pl.ds(i * L, L)] = jnp.zeros((L,), jnp.float32)

    pltpu.sync_copy(zero_v, acc.at[pl.ds(sid * SL, SL)])
    plsc.subcore_barrier()


def _acc_to_out(acc, out_h):
    cid = lax.axis_index("c")
    sid = lax.axis_index("s")
    plsc.subcore_barrier()
    pltpu.sync_copy(acc.at[pl.ds(sid * SL, SL)],
                    out_h.at[pl.ds(cid * NP + sid * SL, SL)])


@functools.partial(
    pl.kernel,
    out_type=[jax.ShapeDtypeStruct((NC * NP,), jnp.float32),
              jax.ShapeDtypeStruct((EP,), jnp.int32)],
    mesh=_MESH,
    compiler_params=_PARAMS,
    scratch_types=[
        [pltpu.VMEM((EC,), jnp.int32)] * 2,
        [pltpu.VMEM((EC,), jnp.int32)] * 2,
        [pltpu.VMEM((EC,), jnp.float32)] * 2,
        [pltpu.VMEM((EC,), jnp.int32)] * 2,
        [pltpu.VMEM((EC,), jnp.int32)] * 2,
        [pltpu.VMEM((EC,), jnp.int32)] * 2,
        [pltpu.VMEM((EC,), jnp.float32)] * 2,
        [pltpu.VMEM((EC,), jnp.int32)] * 2,
        pltpu.VMEM((96,), jnp.float32),
        pltpu.VMEM((SL,), jnp.float32),
        pltpu.VMEM_SHARED((NP,), jnp.float32),
        [pltpu.SemaphoreType.DMA] * 2,
        [pltpu.SemaphoreType.DMA] * 2,
        [pltpu.SemaphoreType.DMA] * 2,
        [pltpu.SemaphoreType.DMA] * 2,
    ],
)
def _cn_kernel(z_h, ii_h, jj_h, d_h, rcov_h, cn_out, p_out,
               ii_v, jj_v, d_v, zi_v, zj_v, p_v, val_v, iis_v, rcov_v,
               zero_v, acc, seml, semg, semp, sems):
    wid = _worker_id()
    pltpu.sync_copy(rcov_h, rcov_v)
    _zero_acc(zero_v, acc)

    ebase = wid * EW

    def base(c):
        return ebase + c * EC

    def issue_lin(c, b):
        pltpu.async_copy(ii_h.at[pl.ds(base(c), EC)], ii_v[b], seml[b])
        pltpu.async_copy(jj_h.at[pl.ds(base(c), EC)], jj_v[b], seml[b])
        pltpu.async_copy(d_h.at[pl.ds(base(c), EC)], d_v[b], seml[b])

    def wait_lin(b):
        pltpu.make_async_copy(ii_h.at[pl.ds(0, EC)], ii_v[b], seml[b]).wait()
        pltpu.make_async_copy(jj_h.at[pl.ds(0, EC)], jj_v[b], seml[b]).wait()
        pltpu.make_async_copy(d_h.at[pl.ds(0, EC)], d_v[b], seml[b]).wait()

    def issue_gat(b):
        pltpu.async_copy(z_h.at[ii_v[b]], zi_v[b], semg[b])
        pltpu.async_copy(z_h.at[jj_v[b]], zj_v[b], semg[b])

    def wait_gat(b):
        pltpu.make_async_copy(z_h.at[pl.ds(0, EC)], zi_v[b], semg[b]).wait()
        pltpu.make_async_copy(z_h.at[pl.ds(0, EC)], zj_v[b], semg[b]).wait()

    def wait_pwrite(b):
        pltpu.make_async_copy(p_v[b], p_out.at[pl.ds(0, EC)], semp[b]).wait()

    def wait_scat(b):
        pltpu.make_async_copy(val_v[b], acc.at[iis_v[b]], sems[b]).wait()

    issue_lin(0, 0)
    wait_lin(0)
    issue_gat(0)
    issue_lin(1, 1)

    @pl.loop(0, NCHUNK // 2)
    def _pair(m):
        for b in (0, 1):
            c = 2 * m + b
            o = 1 - b
            if b == 0:
                wait_lin(o)
                issue_gat(o)
            else:
                @pl.when(m < NCHUNK // 2 - 1)
                def _():
                    wait_lin(o)
                    issue_gat(o)
            wait_gat(b)

            @pl.when(m >= 1)
            def _():
                wait_pwrite(b)
                wait_scat(b)

            @pl.loop(0, EC // L)
            def _(i):
                s = pl.ds(i * L, L)
                zi = zi_v[b][s]
                zj = zj_v[b][s]
                ri = plsc.load_gather(rcov_v, [zi])
                rj = plsc.load_gather(rcov_v, [zj])
                rco = D3_K2 * (ri + rj)
                t = jnp.exp(-D3_K1 * (rco / d_v[b][s] - 1.0))
                val_v[b][s] = 1.0 / (1.0 + t)
                p_v[b][s] = zi * N_ELEM + zj
                iis_v[b][s] = ii_v[b][s]

            pltpu.async_copy(p_v[b], p_out.at[pl.ds(base(c), EC)], semp[b])
            pltpu.async_copy(val_v[b], acc.at[iis_v[b]], sems[b], add=True)

            @pl.when(m < NCHUNK // 2 - 1)
            def _():
                issue_lin(c + 2, b)

    wait_pwrite(0)
    wait_pwrite(1)
    wait_scat(0)
    wait_scat(1)
    _acc_to_out(acc, cn_out)


@functools.partial(
    pl.kernel,
    out_type=jax.ShapeDtypeStruct((NP, 8), jnp.float32),
    mesh=_MESH,
    compiler_params=_PARAMS,
    scratch_types=[
        pltpu.VMEM((NB,), jnp.float32),
        pltpu.VMEM((NB,), jnp.float32),
        pltpu.VMEM((NB,), jnp.int32),
        pltpu.VMEM((NB, 8), jnp.float32),
        pltpu.VMEM((480,), jnp.float32),
        pltpu.VMEM((96,), jnp.float32),
        pltpu.VMEM((96,), jnp.float32),
    ],
)
def _gw_kernel(cn_h, z_h, rcn_h, q_h, sq_h, out_h,
               cn0_v, cn1_v, z_v, feat_v, rcn_v, q_v, sq_v):
    wid = _worker_id()
    nb = wid * NB
    pltpu.sync_copy(rcn_h, rcn_v)
    pltpu.sync_copy(q_h, q_v)
    pltpu.sync_copy(sq_h, sq_v)
    pltpu.sync_copy(cn_h.at[pl.ds(nb, NB)], cn0_v)
    pltpu.sync_copy(cn_h.at[pl.ds(NP + nb, NB)], cn1_v)
    pltpu.sync_copy(z_h.at[pl.ds(nb, NB)], z_v)

    iota = lax.iota(jnp.int32, L)
    cols = [jnp.full((L,), k, jnp.int32) for k in range(8)]

    @pl.loop(0, NB // L)
    def _(i):
        s = pl.ds(i * L, L)
        z = z_v[s]
        cn = cn0_v[s] + cn1_v[s]
        zb = z * N_REF
        r = [plsc.load_gather(rcn_v, [zb + k]) for k in range(N_REF)]
        maxcn = r[0]
        for k in range(1, N_REF):
            maxcn = jnp.maximum(maxcn, r[k])
        w = []
        norm = None
        for k in range(N_REF):
            d = cn - r[k]
            wk = jnp.exp(D3_K3 * d * d)
            w.append(wk)
            norm = wk if norm is None else norm + wk
        exc = norm < 1e-30
        safe = jnp.where(exc, 1.0, norm)
        rows = i * L + iota
        for k in range(N_REF):
            gwk = jnp.where(exc, jnp.where(r[k] == maxcn, 1.0, 0.0),
                            w[k] / safe)
            plsc.store_scatter(feat_v, [rows, cols[k]], gwk)
        plsc.store_scatter(feat_v, [rows, cols[5]],
                           plsc.load_gather(q_v, [z]))
        plsc.store_scatter(feat_v, [rows, cols[6]],
                           plsc.load_gather(sq_v, [z]))
        plsc.store_scatter(feat_v, [rows, cols[7]], z.astype(jnp.float32))

    pltpu.sync_copy(feat_v, out_h.at[pl.ds(nb, NB), :])


@functools.partial(
    pl.kernel,
    out_type=jax.ShapeDtypeStruct((NC * NP,), jnp.float32),
    mesh=_MESH,
    compiler_params=_PARAMS,
    scratch_types=[
        [pltpu.VMEM((EC,), jnp.int32)] * 2,
        [pltpu.VMEM((EC,), jnp.int32)] * 2,
        [pltpu.VMEM((EC,), jnp.int32)] * 2,
        [pltpu.VMEM((EC,), jnp.float32)] * 2,
        [pltpu.VMEM((EC, 8), jnp.float32)] * 2,
        [pltpu.VMEM((EC, 8), jnp.float32)] * 2,
        [pltpu.VMEM((EC, 32), jnp.float32)] * 2,
        [pltpu.VMEM((EC,), jnp.float32)] * 2,
        [pltpu.VMEM((EC,), jnp.int32)] * 2,
        pltpu.VMEM((SL,), jnp.float32),
        pltpu.VMEM_SHARED((NP,), jnp.float32),
        [pltpu.SemaphoreType.DMA] * 2,
        [pltpu.SemaphoreType.DMA] * 2,
        [pltpu.SemaphoreType.DMA] * 2,
    ],
)
def _edisp_kernel(ii_h, jj_h, d_h, p_h, feat_h, rc6_h, out_h,
                  ii_v, jj_v, p_v, d_v, wi_v, wj_v, rows_v, val_v, iis_v,
                  zero_v, acc, seml, semg, sems):
    wid = _worker_id()
    _zero_acc(zero_v, acc)

    ebase = wid * EW
    iota = lax.iota(jnp.int32, L)
    cols = [jnp.full((L,), k, jnp.int32) for k in range(32)]

    def base(c):
        return ebase + c * EC

    def issue_lin(c, b):
        pltpu.async_copy(ii_h.at[pl.ds(base(c), EC)], ii_v[b], seml[b])
        pltpu.async_copy(jj_h.at[pl.ds(base(c), EC)], jj_v[b], seml[b])
        pltpu.async_copy(p_h.at[pl.ds(base(c), EC)], p_v[b], seml[b])
        pltpu.async_copy(d_h.at[pl.ds(base(c), EC)], d_v[b], seml[b])

    def wait_lin(b):
        pltpu.make_async_copy(ii_h.at[pl.ds(0, EC)], ii_v[b], seml[b]).wait()
        pltpu.make_async_copy(jj_h.at[pl.ds(0, EC)], jj_v[b], seml[b]).wait()
        pltpu.make_async_copy(p_h.at[pl.ds(0, EC)], p_v[b], seml[b]).wait()
        pltpu.make_async_copy(d_h.at[pl.ds(0, EC)], d_v[b], seml[b]).wait()

    def issue_gat(b):
        pltpu.async_copy(feat_h.at[ii_v[b]], wi_v[b], semg[b])
        pltpu.async_copy(feat_h.at[jj_v[b]], wj_v[b], semg[b])
        pltpu.async_copy(rc6_h.at[p_v[b]], rows_v[b], semg[b])

    def wait_gat(b):
        pltpu.make_async_copy(feat_h.at[pl.ds(0, EC), :], wi_v[b],
                              semg[b]).wait()
        pltpu.make_async_copy(feat_h.at[pl.ds(0, EC), :], wj_v[b],
                              semg[b]).wait()
        pltpu.make_async_copy(rc6_h.at[pl.ds(0, EC), :], rows_v[b],
                              semg[b]).wait()

    def wait_scat(b):
        pltpu.make_async_copy(val_v[b], acc.at[iis_v[b]], sems[b]).wait()

    issue_lin(0, 0)
    wait_lin(0)
    issue_gat(0)
    issue_lin(1, 1)

    @pl.loop(0, NCHUNK // 2)
    def _pair(m):
      for b in (0, 1):
        c = 2 * m + b
        o = 1 - b
        if b == 0:
            wait_lin(o)
            issue_gat(o)
        else:
            @pl.when(m < NCHUNK // 2 - 1)
            def _():
                wait_lin(o)
                issue_gat(o)
        wait_gat(b)

        @pl.when(m >= 1)
        def _():
            wait_scat(b)

        @pl.loop(0, EC // L)
        def _(i):
            s = pl.ds(i * L, L)
            rows = i * L + iota
            wi = [plsc.load_gather(wi_v[b], [rows, cols[a]])
                  for a in range(N_REF)]
            wj = [plsc.load_gather(wj_v[b], [rows, cols[k]])
                  for k in range(N_REF)]
            c6 = None
            for a in range(N_REF):
                ta = None
                for k in range(N_REF):
                    rab = plsc.load_gather(rows_v[b],
                                           [rows, cols[a * N_REF + k]])
                    t = wj[k] * rab
                    ta = t if ta is None else ta + t
                t = wi[a] * ta
                c6 = t if c6 is None else c6 + t
            qi = plsc.load_gather(wi_v[b], [rows, cols[5]])
            qj = plsc.load_gather(wj_v[b], [rows, cols[5]])
            sqi = plsc.load_gather(wi_v[b], [rows, cols[6]])
            sqj = plsc.load_gather(wj_v[b], [rows, cols[6]])
            qq = 3.0 * qi * qj
            c8 = c6 * qq
            rr = D3_A1 * SQRT3 * sqi * sqj + D3_A2
            r = d_v[b][s]
            r2 = r * r
            r6 = r2 * r2 * r2
            r8 = r6 * r2
            rr2 = rr * rr
            rr6 = rr2 * rr2 * rr2
            rr8 = rr6 * rr2
            val_v[b][s] = -0.5 * (D3_S6 * c6 / (r6 + rr6)
                                  + D3_S8 * c8 / (r8 + rr8))
            iis_v[b][s] = ii_v[b][s]

        pltpu.async_copy(val_v[b], acc.at[iis_v[b]], sems[b], add=True)

        @pl.when(m < NCHUNK // 2 - 1)
        def _():
            issue_lin(c + 2, b)

    wait_scat(0)
    wait_scat(1)
    _acc_to_out(acc, out_h)


def _add_body(x_ref, o_ref):
    o_ref[...] = x_ref[0] + x_ref[1]


_add_call = pl.pallas_call(
    _add_body,
    out_shape=jax.ShapeDtypeStruct((NP // 128, 128), jnp.float32),
)


def kernel(atomic_numbers, distances, idx_i, idx_j,
           d3_rcov, d3_rcn, d3_rc6, d3_r2r4):
    z = atomic_numbers.astype(jnp.int32)
    ii = idx_i.astype(jnp.int32)
    jj = idx_j.astype(jnp.int32)
    dist = distances.astype(jnp.float32)

    zp = jnp.pad(z, (0, NP - N_NODES))
    pad_e = EP - N_EDGES
    iip = jnp.pad(ii, (0, pad_e), constant_values=N_NODES)
    jjp = jnp.pad(jj, (0, pad_e))
    dp = jnp.pad(dist, (0, pad_e), constant_values=1.0)

    rcov96 = jnp.pad(d3_rcov.astype(jnp.float32), (0, 96 - N_ELEM))
    rcn480 = jnp.pad(d3_rcn.astype(jnp.float32).reshape(-1),
                     (0, 480 - N_ELEM * N_REF))
    q96 = jnp.pad(d3_r2r4.astype(jnp.float32), (0, 96 - N_ELEM))
    sq96 = jnp.sqrt(q96)
    rc6p = jnp.pad(
        d3_rc6.astype(jnp.float32).reshape(N_ELEM * N_ELEM, N_REF * N_REF),
        ((0, 0), (0, 32 - N_REF * N_REF)))

    cn_parts, pidx = _cn_kernel(zp, iip, jjp, dp, rcov96)
    feat = _gw_kernel(cn_parts, zp, rcn480, q96, sq96)
    e_parts = _edisp_kernel(iip, jjp, dp, pidx, feat, rc6p)
    edisp = _add_call(e_parts.reshape(NC, NP // 128, 128))
    return edisp.reshape(NP)[:N_NODES]


# speedup vs baseline: 104.3740x; 1.4660x over previous
import functools

import jax
import jax.numpy as jnp
from jax import lax
from jax.experimental import pallas as pl
from jax.experimental.pallas import tpu as pltpu
from jax.experimental.pallas import tpu_sc as plsc

N_NODES = 100000
N_EDGES = 1600000
N_ELEM = 95
N_REF = 5

NC = 2
NS = 16
L = 16
NW = NC * NS

NB = 3136
NP = NB * NW
SL = NP // NS

EC = 1024
NCHUNK = 50
EW = EC * NCHUNK
EP = EW * NW

D3_K1 = 16.0
D3_K2 = 4.0 / 3.0
D3_K3 = -4.0
D3_S6 = 1.0
D3_S8 = 0.9171
D3_A1 = 0.3385
D3_A2 = 2.883
SQRT3 = 3.0 ** 0.5

_MESH = plsc.VectorSubcoreMesh(core_axis_name="c", subcore_axis_name="s",
                               num_cores=NC, num_subcores=NS)
_PARAMS = pltpu.CompilerParams(needs_layout_passes=False,
                               use_tc_tiling_on_sc=False)


def _worker_id():
    return lax.axis_index("c") * NS + lax.axis_index("s")


def _zero_acc(zero_v, acc):
    sid = lax.axis_index("s")

    @pl.loop(0, SL // L)
    def _(i):
        zero_v[pl.ds(i * L, L)] = jnp.zeros((L,), jnp.float32)

    pltpu.sync_copy(zero_v, acc.at[pl.ds(sid * SL, SL)])
    plsc.subcore_barrier()


def _acc_to_out(acc, out_h):
    cid = lax.axis_index("c")
    sid = lax.axis_index("s")
    plsc.subcore_barrier()
    pltpu.sync_copy(acc.at[pl.ds(sid * SL, SL)],
                    out_h.at[pl.ds(cid * NP + sid * SL, SL)])


@functools.partial(
    pl.kernel,
    out_type=[jax.ShapeDtypeStruct((NC * NP,), jnp.float32),
              jax.ShapeDtypeStruct((EP,), jnp.int32)],
    mesh=_MESH,
    compiler_params=_PARAMS,
    scratch_types=[
        [pltpu.VMEM((EC,), jnp.int32)] * 2,
        [pltpu.VMEM((EC,), jnp.int32)] * 2,
        [pltpu.VMEM((EC,), jnp.float32)] * 2,
        [pltpu.VMEM((EC,), jnp.int32)] * 2,
        [pltpu.VMEM((EC,), jnp.int32)] * 2,
        [pltpu.VMEM((EC,), jnp.int32)] * 2,
        [pltpu.VMEM((EC,), jnp.float32)] * 2,
        [pltpu.VMEM((EC,), jnp.int32)] * 2,
        pltpu.VMEM((96,), jnp.float32),
        pltpu.VMEM((SL,), jnp.float32),
        pltpu.VMEM_SHARED((NP,), jnp.float32),
        [pltpu.SemaphoreType.DMA] * 2,
        [pltpu.SemaphoreType.DMA] * 2,
        [pltpu.SemaphoreType.DMA] * 2,
        [pltpu.SemaphoreType.DMA] * 2,
    ],
)
def _cn_kernel(z_h, ii_h, jj_h, d_h, rcov_h, cn_out, p_out,
               ii_v, jj_v, d_v, zi_v, zj_v, p_v, val_v, iis_v, rcov_v,
               zero_v, acc, seml, semg, semp, sems):
    wid = _worker_id()
    pltpu.sync_copy(rcov_h, rcov_v)
    _zero_acc(zero_v, acc)

    ebase = wid * EW

    def base(c):
        return ebase + c * EC

    def issue_lin(c, b):
        pltpu.async_copy(ii_h.at[pl.ds(base(c), EC)], ii_v[b], seml[b])
        pltpu.async_copy(jj_h.at[pl.ds(base(c), EC)], jj_v[b], seml[b])
        pltpu.async_copy(d_h.at[pl.ds(base(c), EC)], d_v[b], seml[b])

    def wait_lin(b):
        pltpu.make_async_copy(ii_h.at[pl.ds(0, EC)], ii_v[b], seml[b]).wait()
        pltpu.make_async_copy(jj_h.at[pl.ds(0, EC)], jj_v[b], seml[b]).wait()
        pltpu.make_async_copy(d_h.at[pl.ds(0, EC)], d_v[b], seml[b]).wait()

    def issue_gat(b):
        pltpu.async_copy(z_h.at[ii_v[b]], zi_v[b], semg[b])
        pltpu.async_copy(z_h.at[jj_v[b]], zj_v[b], semg[b])

    def wait_gat(b):
        pltpu.make_async_copy(z_h.at[pl.ds(0, EC)], zi_v[b], semg[b]).wait()
        pltpu.make_async_copy(z_h.at[pl.ds(0, EC)], zj_v[b], semg[b]).wait()

    def wait_pwrite(b):
        pltpu.make_async_copy(p_v[b], p_out.at[pl.ds(0, EC)], semp[b]).wait()

    def wait_scat(b):
        pltpu.make_async_copy(val_v[b], acc.at[iis_v[b]], sems[b]).wait()

    issue_lin(0, 0)
    wait_lin(0)
    issue_gat(0)
    issue_lin(1, 1)

    @pl.loop(0, NCHUNK // 2)
    def _pair(m):
        for b in (0, 1):
            c = 2 * m + b
            o = 1 - b
            if b == 0:
                wait_lin(o)
                issue_gat(o)
            else:
                @pl.when(m < NCHUNK // 2 - 1)
                def _():
                    wait_lin(o)
                    issue_gat(o)
            wait_gat(b)

            @pl.when(m >= 1)
            def _():
                wait_pwrite(b)
                wait_scat(b)

            @pl.loop(0, EC // L)
            def _(i):
                s = pl.ds(i * L, L)
                zi = zi_v[b][s]
                zj = zj_v[b][s]
                ri = plsc.load_gather(rcov_v, [zi])
                rj = plsc.load_gather(rcov_v, [zj])
                rco = D3_K2 * (ri + rj)
                t = jnp.exp(-D3_K1 * (rco / d_v[b][s] - 1.0))
                val_v[b][s] = 1.0 / (1.0 + t)
                p_v[b][s] = zi * N_ELEM + zj
                iis_v[b][s] = ii_v[b][s]

            pltpu.async_copy(p_v[b], p_out.at[pl.ds(base(c), EC)], semp[b])
            pltpu.async_copy(val_v[b], acc.at[iis_v[b]], sems[b], add=True)

            @pl.when(m < NCHUNK // 2 - 1)
            def _():
                issue_lin(c + 2, b)

    wait_pwrite(0)
    wait_pwrite(1)
    wait_scat(0)
    wait_scat(1)
    _acc_to_out(acc, cn_out)


@functools.partial(
    pl.kernel,
    out_type=jax.ShapeDtypeStruct((NP, 8), jnp.float32),
    mesh=_MESH,
    compiler_params=_PARAMS,
    scratch_types=[
        pltpu.VMEM((NB,), jnp.float32),
        pltpu.VMEM((NB,), jnp.float32),
        pltpu.VMEM((NB,), jnp.int32),
        pltpu.VMEM((NB, 8), jnp.float32),
        pltpu.VMEM((480,), jnp.float32),
        pltpu.VMEM((96,), jnp.float32),
        pltpu.VMEM((96,), jnp.float32),
    ],
)
def _gw_kernel(cn_h, z_h, rcn_h, q_h, sq_h, out_h,
               cn0_v, cn1_v, z_v, feat_v, rcn_v, q_v, sq_v):
    wid = _worker_id()
    nb = wid * NB
    pltpu.sync_copy(rcn_h, rcn_v)
    pltpu.sync_copy(q_h, q_v)
    pltpu.sync_copy(sq_h, sq_v)
    pltpu.sync_copy(cn_h.at[pl.ds(nb, NB)], cn0_v)
    pltpu.sync_copy(cn_h.at[pl.ds(NP + nb, NB)], cn1_v)
    pltpu.sync_copy(z_h.at[pl.ds(nb, NB)], z_v)

    iota = lax.iota(jnp.int32, L)
    cols = [jnp.full((L,), k, jnp.int32) for k in range(8)]

    @pl.loop(0, NB // L)
    def _(i):
        s = pl.ds(i * L, L)
        z = z_v[s]
        cn = cn0_v[s] + cn1_v[s]
        zb = z * N_REF
        r = [plsc.load_gather(rcn_v, [zb + k]) for k in range(N_REF)]
        maxcn = r[0]
        for k in range(1, N_REF):
            maxcn = jnp.maximum(maxcn, r[k])
        w = []
        norm = None
        for k in range(N_REF):
            d = cn - r[k]
            wk = jnp.exp(D3_K3 * d * d)
            w.append(wk)
            norm = wk if norm is None else norm + wk
        exc = norm < 1e-30
        safe = jnp.where(exc, 1.0, norm)
        rows = i * L + iota
        for k in range(N_REF):
            gwk = jnp.where(exc, jnp.where(r[k] == maxcn, 1.0, 0.0),
                            w[k] / safe)
            plsc.store_scatter(feat_v, [rows, cols[k]], gwk)
        plsc.store_scatter(feat_v, [rows, cols[5]],
                           plsc.load_gather(q_v, [z]))
        plsc.store_scatter(feat_v, [rows, cols[6]],
                           plsc.load_gather(sq_v, [z]))
        plsc.store_scatter(feat_v, [rows, cols[7]], z.astype(jnp.float32))

    pltpu.sync_copy(feat_v, out_h.at[pl.ds(nb, NB), :])


@functools.partial(
    pl.kernel,
    out_type=jax.ShapeDtypeStruct((NC * NP,), jnp.float32),
    mesh=_MESH,
    compiler_params=_PARAMS,
    scratch_types=[
        [pltpu.VMEM((EC,), jnp.int32)] * 2,
        [pltpu.VMEM((EC,), jnp.int32)] * 2,
        [pltpu.VMEM((EC,), jnp.int32)] * 2,
        [pltpu.VMEM((EC,), jnp.float32)] * 2,
        [pltpu.VMEM((EC, 8), jnp.float32)] * 2,
        [pltpu.VMEM((EC, 8), jnp.float32)] * 2,
        [pltpu.VMEM((EC, 16), jnp.int32)] * 2,
        [pltpu.VMEM((EC,), jnp.float32)] * 2,
        [pltpu.VMEM((EC,), jnp.int32)] * 2,
        pltpu.VMEM((SL,), jnp.float32),
        pltpu.VMEM_SHARED((NP,), jnp.float32),
        [pltpu.SemaphoreType.DMA] * 2,
        [pltpu.SemaphoreType.DMA] * 2,
        [pltpu.SemaphoreType.DMA] * 2,
    ],
)
def _edisp_kernel(ii_h, jj_h, d_h, p_h, feat_h, rc6_h, out_h,
                  ii_v, jj_v, p_v, d_v, wi_v, wj_v, rows_v, val_v, iis_v,
                  zero_v, acc, seml, semg, sems):
    wid = _worker_id()
    _zero_acc(zero_v, acc)

    ebase = wid * EW
    iota = lax.iota(jnp.int32, L)
    cols = [jnp.full((L,), k, jnp.int32) for k in range(16)]

    def base(c):
        return ebase + c * EC

    def issue_lin(c, b):
        pltpu.async_copy(ii_h.at[pl.ds(base(c), EC)], ii_v[b], seml[b])
        pltpu.async_copy(jj_h.at[pl.ds(base(c), EC)], jj_v[b], seml[b])
        pltpu.async_copy(p_h.at[pl.ds(base(c), EC)], p_v[b], seml[b])
        pltpu.async_copy(d_h.at[pl.ds(base(c), EC)], d_v[b], seml[b])

    def wait_lin(b):
        pltpu.make_async_copy(ii_h.at[pl.ds(0, EC)], ii_v[b], seml[b]).wait()
        pltpu.make_async_copy(jj_h.at[pl.ds(0, EC)], jj_v[b], seml[b]).wait()
        pltpu.make_async_copy(p_h.at[pl.ds(0, EC)], p_v[b], seml[b]).wait()
        pltpu.make_async_copy(d_h.at[pl.ds(0, EC)], d_v[b], seml[b]).wait()

    def issue_gat(b):
        pltpu.async_copy(feat_h.at[ii_v[b]], wi_v[b], semg[b])
        pltpu.async_copy(feat_h.at[jj_v[b]], wj_v[b], semg[b])
        pltpu.async_copy(rc6_h.at[p_v[b]], rows_v[b], semg[b])

    def wait_gat(b):
        pltpu.make_async_copy(feat_h.at[pl.ds(0, EC), :], wi_v[b],
                              semg[b]).wait()
        pltpu.make_async_copy(feat_h.at[pl.ds(0, EC), :], wj_v[b],
                              semg[b]).wait()
        pltpu.make_async_copy(rc6_h.at[pl.ds(0, EC), :], rows_v[b],
                              semg[b]).wait()

    def wait_scat(b):
        pltpu.make_async_copy(val_v[b], acc.at[iis_v[b]], sems[b]).wait()

    issue_lin(0, 0)
    wait_lin(0)
    issue_gat(0)
    issue_lin(1, 1)

    @pl.loop(0, NCHUNK // 2)
    def _pair(m):
      for b in (0, 1):
        c = 2 * m + b
        o = 1 - b
        if b == 0:
            wait_lin(o)
            issue_gat(o)
        else:
            @pl.when(m < NCHUNK // 2 - 1)
            def _():
                wait_lin(o)
                issue_gat(o)
        wait_gat(b)

        @pl.when(m >= 1)
        def _():
            wait_scat(b)

        @pl.loop(0, EC // L)
        def _(i):
            s = pl.ds(i * L, L)
            rows = i * L + iota
            wi = [plsc.load_gather(wi_v[b], [rows, cols[a]])
                  for a in range(N_REF)]
            wj = [plsc.load_gather(wj_v[b], [rows, cols[k]])
                  for k in range(N_REF)]
            c6 = None
            for w in range(13):
                word = plsc.load_gather(rows_v[b], [rows, cols[w]])
                bfp = plsc.bitcast(word, jnp.bfloat16)
                ev, od = plsc.unpack(bfp, format=plsc.PackFormat.INTERLEAVED)
                a0, b0 = divmod(2 * w, 5)
                t = (wi[a0] * wj[b0]) * ev
                c6 = t if c6 is None else c6 + t
                if 2 * w + 1 < N_REF * N_REF:
                    a1, b1 = divmod(2 * w + 1, 5)
                    c6 = c6 + (wi[a1] * wj[b1]) * od
            qi = plsc.load_gather(wi_v[b], [rows, cols[5]])
            qj = plsc.load_gather(wj_v[b], [rows, cols[5]])
            sqi = plsc.load_gather(wi_v[b], [rows, cols[6]])
            sqj = plsc.load_gather(wj_v[b], [rows, cols[6]])
            qq = 3.0 * qi * qj
            c8 = c6 * qq
            rr = D3_A1 * SQRT3 * sqi * sqj + D3_A2
            r = d_v[b][s]
            r2 = r * r
            r6 = r2 * r2 * r2
            r8 = r6 * r2
            rr2 = rr * rr
            rr6 = rr2 * rr2 * rr2
            rr8 = rr6 * rr2
            val_v[b][s] = -0.5 * (D3_S6 * c6 / (r6 + rr6)
                                  + D3_S8 * c8 / (r8 + rr8))
            iis_v[b][s] = ii_v[b][s]

        pltpu.async_copy(val_v[b], acc.at[iis_v[b]], sems[b], add=True)

        @pl.when(m < NCHUNK // 2 - 1)
        def _():
            issue_lin(c + 2, b)

    wait_scat(0)
    wait_scat(1)
    _acc_to_out(acc, out_h)


def _add_body(x_ref, o_ref):
    o_ref[...] = x_ref[0] + x_ref[1]


_add_call = pl.pallas_call(
    _add_body,
    out_shape=jax.ShapeDtypeStruct((NP // 128, 128), jnp.float32),
)


def kernel(atomic_numbers, distances, idx_i, idx_j,
           d3_rcov, d3_rcn, d3_rc6, d3_r2r4):
    z = atomic_numbers.astype(jnp.int32)
    ii = idx_i.astype(jnp.int32)
    jj = idx_j.astype(jnp.int32)
    dist = distances.astype(jnp.float32)

    zp = jnp.pad(z, (0, NP - N_NODES))
    pad_e = EP - N_EDGES
    iip = jnp.pad(ii, (0, pad_e), constant_values=N_NODES)
    jjp = jnp.pad(jj, (0, pad_e))
    dp = jnp.pad(dist, (0, pad_e), constant_values=1.0)

    rcov96 = jnp.pad(d3_rcov.astype(jnp.float32), (0, 96 - N_ELEM))
    rcn480 = jnp.pad(d3_rcn.astype(jnp.float32).reshape(-1),
                     (0, 480 - N_ELEM * N_REF))
    q96 = jnp.pad(d3_r2r4.astype(jnp.float32), (0, 96 - N_ELEM))
    sq96 = jnp.sqrt(q96)
    rc6b = jnp.pad(
        d3_rc6.astype(jnp.bfloat16).reshape(N_ELEM * N_ELEM, N_REF * N_REF),
        ((0, 0), (0, 32 - N_REF * N_REF)))
    rc6p = lax.bitcast_convert_type(
        rc6b.reshape(N_ELEM * N_ELEM, 16, 2), jnp.int32)

    cn_parts, pidx = _cn_kernel(zp, iip, jjp, dp, rcov96)
    feat = _gw_kernel(cn_parts, zp, rcn480, q96, sq96)
    e_parts = _edisp_kernel(iip, jjp, dp, pidx, feat, rc6p)
    edisp = _add_call(e_parts.reshape(NC, NP // 128, 128))
    return edisp.reshape(NP)[:N_NODES]


# speedup vs baseline: 157.7339x; 1.5112x over previous
import functools

import jax
import jax.numpy as jnp
from jax import lax
from jax.experimental import pallas as pl
from jax.experimental.pallas import tpu as pltpu
from jax.experimental.pallas import tpu_sc as plsc

N_NODES = 100000
N_EDGES = 1600000
N_ELEM = 95
N_REF = 5

NC = 2
NS = 16
L = 16
NW = NC * NS

NB = 3136
NP = NB * NW
SL = NP // NS

EC = 1024
NCHUNK0 = 64
NCHUNK1 = 34
EP = EC * NS * (NCHUNK0 + NCHUNK1)

D3_K1 = 16.0
D3_K2 = 4.0 / 3.0
D3_K3 = -4.0
D3_S6 = 1.0
D3_S8 = 0.9171
D3_A1 = 0.3385
D3_A2 = 2.883
SQRT3 = 3.0 ** 0.5

_MESH = plsc.VectorSubcoreMesh(core_axis_name="c", subcore_axis_name="s",
                               num_cores=NC, num_subcores=NS)
_PARAMS = pltpu.CompilerParams(needs_layout_passes=False,
                               use_tc_tiling_on_sc=False)


def _edge_split():
    cid = lax.axis_index("c")
    sid = lax.axis_index("s")
    ebase = jnp.where(cid == 0, sid * (NCHUNK0 * EC),
                      NS * (NCHUNK0 * EC) + sid * (NCHUNK1 * EC))
    half = jnp.where(cid == 0, NCHUNK0 // 2, NCHUNK1 // 2)
    return ebase, half


def _worker_id():
    return lax.axis_index("c") * NS + lax.axis_index("s")


def _zero_acc(zero_v, acc):
    sid = lax.axis_index("s")

    @pl.loop(0, SL // L)
    def _(i):
        zero_v[pl.ds(i * L, L)] = jnp.zeros((L,), jnp.float32)

    pltpu.sync_copy(zero_v, acc.at[pl.ds(sid * SL, SL)])
    plsc.subcore_barrier()


def _acc_to_out(acc, out_h):
    cid = lax.axis_index("c")
    sid = lax.axis_index("s")
    plsc.subcore_barrier()
    pltpu.sync_copy(acc.at[pl.ds(sid * SL, SL)],
                    out_h.at[pl.ds(cid * NP + sid * SL, SL)])


@functools.partial(
    pl.kernel,
    out_type=[jax.ShapeDtypeStruct((NC * NP,), jnp.float32),
              jax.ShapeDtypeStruct((EP,), jnp.int32)],
    mesh=_MESH,
    compiler_params=_PARAMS,
    scratch_types=[
        [pltpu.VMEM((EC,), jnp.int32)] * 2,
        [pltpu.VMEM((EC,), jnp.int32)] * 2,
        [pltpu.VMEM((EC,), jnp.float32)] * 2,
        [pltpu.VMEM((EC,), jnp.int32)] * 2,
        [pltpu.VMEM((EC,), jnp.int32)] * 2,
        [pltpu.VMEM((EC,), jnp.int32)] * 2,
        [pltpu.VMEM((EC,), jnp.float32)] * 2,
        [pltpu.VMEM((EC,), jnp.int32)] * 2,
        pltpu.VMEM((96,), jnp.float32),
        pltpu.VMEM((SL,), jnp.float32),
        pltpu.VMEM_SHARED((NP,), jnp.float32),
        [pltpu.SemaphoreType.DMA] * 2,
        [pltpu.SemaphoreType.DMA] * 2,
        [pltpu.SemaphoreType.DMA] * 2,
        [pltpu.SemaphoreType.DMA] * 2,
    ],
)
def _cn_kernel(z_h, ii_h, jj_h, d_h, rcov_h, cn_out, p_out,
               ii_v, jj_v, d_v, zi_v, zj_v, p_v, val_v, iis_v, rcov_v,
               zero_v, acc, seml, semg, semp, sems):
    pltpu.sync_copy(rcov_h, rcov_v)
    _zero_acc(zero_v, acc)

    ebase, half = _edge_split()

    def base(c):
        return ebase + c * EC

    def issue_lin(c, b):
        pltpu.async_copy(ii_h.at[pl.ds(base(c), EC)], ii_v[b], seml[b])
        pltpu.async_copy(jj_h.at[pl.ds(base(c), EC)], jj_v[b], seml[b])
        pltpu.async_copy(d_h.at[pl.ds(base(c), EC)], d_v[b], seml[b])

    def wait_lin(b):
        pltpu.make_async_copy(ii_h.at[pl.ds(0, EC)], ii_v[b], seml[b]).wait()
        pltpu.make_async_copy(jj_h.at[pl.ds(0, EC)], jj_v[b], seml[b]).wait()
        pltpu.make_async_copy(d_h.at[pl.ds(0, EC)], d_v[b], seml[b]).wait()

    def issue_gat(b):
        pltpu.async_copy(z_h.at[ii_v[b]], zi_v[b], semg[b])
        pltpu.async_copy(z_h.at[jj_v[b]], zj_v[b], semg[b])

    def wait_gat(b):
        pltpu.make_async_copy(z_h.at[pl.ds(0, EC)], zi_v[b], semg[b]).wait()
        pltpu.make_async_copy(z_h.at[pl.ds(0, EC)], zj_v[b], semg[b]).wait()

    def wait_pwrite(b):
        pltpu.make_async_copy(p_v[b], p_out.at[pl.ds(0, EC)], semp[b]).wait()

    def wait_scat(b):
        pltpu.make_async_copy(val_v[b], acc.at[iis_v[b]], sems[b]).wait()

    issue_lin(0, 0)
    wait_lin(0)
    issue_gat(0)
    issue_lin(1, 1)

    @pl.loop(0, half)
    def _pair(m):
        for b in (0, 1):
            c = 2 * m + b
            o = 1 - b
            if b == 0:
                wait_lin(o)
                issue_gat(o)
            else:
                @pl.when(m < half - 1)
                def _():
                    wait_lin(o)
                    issue_gat(o)
            wait_gat(b)

            @pl.when(m >= 1)
            def _():
                wait_pwrite(b)
                wait_scat(b)

            @pl.loop(0, EC // L)
            def _(i):
                s = pl.ds(i * L, L)
                zi = zi_v[b][s]
                zj = zj_v[b][s]
                ri = plsc.load_gather(rcov_v, [zi])
                rj = plsc.load_gather(rcov_v, [zj])
                rco = D3_K2 * (ri + rj)
                t = jnp.exp(-D3_K1 * (rco / d_v[b][s] - 1.0))
                val_v[b][s] = 1.0 / (1.0 + t)
                p_v[b][s] = zi * N_ELEM + zj
                iis_v[b][s] = ii_v[b][s]

            pltpu.async_copy(p_v[b], p_out.at[pl.ds(base(c), EC)], semp[b])
            pltpu.async_copy(val_v[b], acc.at[iis_v[b]], sems[b], add=True)

            @pl.when(m < half - 1)
            def _():
                issue_lin(c + 2, b)

    wait_pwrite(0)
    wait_pwrite(1)
    wait_scat(0)
    wait_scat(1)
    _acc_to_out(acc, cn_out)


@functools.partial(
    pl.kernel,
    out_type=jax.ShapeDtypeStruct((NP, 8), jnp.float32),
    mesh=_MESH,
    compiler_params=_PARAMS,
    scratch_types=[
        pltpu.VMEM((NB,), jnp.float32),
        pltpu.VMEM((NB,), jnp.float32),
        pltpu.VMEM((NB,), jnp.int32),
        pltpu.VMEM((NB, 8), jnp.float32),
        pltpu.VMEM((480,), jnp.float32),
        pltpu.VMEM((96,), jnp.float32),
        pltpu.VMEM((96,), jnp.float32),
    ],
)
def _gw_kernel(cn_h, z_h, rcn_h, q_h, sq_h, out_h,
               cn0_v, cn1_v, z_v, feat_v, rcn_v, q_v, sq_v):
    wid = _worker_id()
    nb = wid * NB
    pltpu.sync_copy(rcn_h, rcn_v)
    pltpu.sync_copy(q_h, q_v)
    pltpu.sync_copy(sq_h, sq_v)
    pltpu.sync_copy(cn_h.at[pl.ds(nb, NB)], cn0_v)
    pltpu.sync_copy(cn_h.at[pl.ds(NP + nb, NB)], cn1_v)
    pltpu.sync_copy(z_h.at[pl.ds(nb, NB)], z_v)

    iota = lax.iota(jnp.int32, L)
    cols = [jnp.full((L,), k, jnp.int32) for k in range(8)]

    @pl.loop(0, NB // L)
    def _(i):
        s = pl.ds(i * L, L)
        z = z_v[s]
        cn = cn0_v[s] + cn1_v[s]
        zb = z * N_REF
        r = [plsc.load_gather(rcn_v, [zb + k]) for k in range(N_REF)]
        maxcn = r[0]
        for k in range(1, N_REF):
            maxcn = jnp.maximum(maxcn, r[k])
        w = []
        norm = None
        for k in range(N_REF):
            d = cn - r[k]
            wk = jnp.exp(D3_K3 * d * d)
            w.append(wk)
            norm = wk if norm is None else norm + wk
        exc = norm < 1e-30
        safe = jnp.where(exc, 1.0, norm)
        rows = i * L + iota
        for k in range(N_REF):
            gwk = jnp.where(exc, jnp.where(r[k] == maxcn, 1.0, 0.0),
                            w[k] / safe)
            plsc.store_scatter(feat_v, [rows, cols[k]], gwk)
        plsc.store_scatter(feat_v, [rows, cols[5]],
                           plsc.load_gather(q_v, [z]))
        plsc.store_scatter(feat_v, [rows, cols[6]],
                           plsc.load_gather(sq_v, [z]))
        plsc.store_scatter(feat_v, [rows, cols[7]], z.astype(jnp.float32))

    pltpu.sync_copy(feat_v, out_h.at[pl.ds(nb, NB), :])


@functools.partial(
    pl.kernel,
    out_type=jax.ShapeDtypeStruct((NC * NP,), jnp.float32),
    mesh=_MESH,
    compiler_params=_PARAMS,
    scratch_types=[
        [pltpu.VMEM((EC,), jnp.int32)] * 2,
        [pltpu.VMEM((EC,), jnp.int32)] * 2,
        [pltpu.VMEM((EC,), jnp.int32)] * 2,
        [pltpu.VMEM((EC,), jnp.float32)] * 2,
        [pltpu.VMEM((EC, 8), jnp.float32)] * 2,
        [pltpu.VMEM((EC, 8), jnp.float32)] * 2,
        [pltpu.VMEM((EC, 16), jnp.int32)] * 2,
        [pltpu.VMEM((EC,), jnp.float32)] * 2,
        [pltpu.VMEM((EC,), jnp.int32)] * 2,
        pltpu.VMEM((SL,), jnp.float32),
        pltpu.VMEM_SHARED((NP,), jnp.float32),
        [pltpu.SemaphoreType.DMA] * 2,
        [pltpu.SemaphoreType.DMA] * 2,
        [pltpu.SemaphoreType.DMA] * 2,
    ],
)
def _edisp_kernel(ii_h, jj_h, d_h, p_h, feat_h, rc6_h, out_h,
                  ii_v, jj_v, p_v, d_v, wi_v, wj_v, rows_v, val_v, iis_v,
                  zero_v, acc, seml, semg, sems):
    _zero_acc(zero_v, acc)

    ebase, half = _edge_split()
    iota = lax.iota(jnp.int32, L)
    cols = [jnp.full((L,), k, jnp.int32) for k in range(16)]

    def base(c):
        return ebase + c * EC

    def issue_lin(c, b):
        pltpu.async_copy(ii_h.at[pl.ds(base(c), EC)], ii_v[b], seml[b])
        pltpu.async_copy(jj_h.at[pl.ds(base(c), EC)], jj_v[b], seml[b])
        pltpu.async_copy(p_h.at[pl.ds(base(c), EC)], p_v[b], seml[b])
        pltpu.async_copy(d_h.at[pl.ds(base(c), EC)], d_v[b], seml[b])

    def wait_lin(b):
        pltpu.make_async_copy(ii_h.at[pl.ds(0, EC)], ii_v[b], seml[b]).wait()
        pltpu.make_async_copy(jj_h.at[pl.ds(0, EC)], jj_v[b], seml[b]).wait()
        pltpu.make_async_copy(p_h.at[pl.ds(0, EC)], p_v[b], seml[b]).wait()
        pltpu.make_async_copy(d_h.at[pl.ds(0, EC)], d_v[b], seml[b]).wait()

    def issue_gat(b):
        pltpu.async_copy(feat_h.at[ii_v[b]], wi_v[b], semg[b])
        pltpu.async_copy(feat_h.at[jj_v[b]], wj_v[b], semg[b])
        pltpu.async_copy(rc6_h.at[p_v[b]], rows_v[b], semg[b])

    def wait_gat(b):
        pltpu.make_async_copy(feat_h.at[pl.ds(0, EC), :], wi_v[b],
                              semg[b]).wait()
        pltpu.make_async_copy(feat_h.at[pl.ds(0, EC), :], wj_v[b],
                              semg[b]).wait()
        pltpu.make_async_copy(rc6_h.at[pl.ds(0, EC), :], rows_v[b],
                              semg[b]).wait()

    def wait_scat(b):
        pltpu.make_async_copy(val_v[b], acc.at[iis_v[b]], sems[b]).wait()

    issue_lin(0, 0)
    wait_lin(0)
    issue_gat(0)
    issue_lin(1, 1)

    @pl.loop(0, half)
    def _pair(m):
      for b in (0, 1):
        c = 2 * m + b
        o = 1 - b
        if b == 0:
            wait_lin(o)
            issue_gat(o)
        else:
            @pl.when(m < half - 1)
            def _():
                wait_lin(o)
                issue_gat(o)
        wait_gat(b)

        @pl.when(m >= 1)
        def _():
            wait_scat(b)

        @pl.loop(0, EC // L)
        def _(i):
            s = pl.ds(i * L, L)
            rows = i * L + iota
            wi = [plsc.load_gather(wi_v[b], [rows, cols[a]])
                  for a in range(N_REF)]
            wj = [plsc.load_gather(wj_v[b], [rows, cols[k]])
                  for k in range(N_REF)]
            c6 = None
            for w in range(13):
                word = plsc.load_gather(rows_v[b], [rows, cols[w]])
                bfp = plsc.bitcast(word, jnp.bfloat16)
                ev, od = plsc.unpack(bfp, format=plsc.PackFormat.INTERLEAVED)
                a0, b0 = divmod(2 * w, 5)
                t = (wi[a0] * wj[b0]) * ev
                c6 = t if c6 is None else c6 + t
                if 2 * w + 1 < N_REF * N_REF:
                    a1, b1 = divmod(2 * w + 1, 5)
                    c6 = c6 + (wi[a1] * wj[b1]) * od
            qi = plsc.load_gather(wi_v[b], [rows, cols[5]])
            qj = plsc.load_gather(wj_v[b], [rows, cols[5]])
            sqi = plsc.load_gather(wi_v[b], [rows, cols[6]])
            sqj = plsc.load_gather(wj_v[b], [rows, cols[6]])
            qq = 3.0 * qi * qj
            c8 = c6 * qq
            rr = D3_A1 * SQRT3 * sqi * sqj + D3_A2
            r = d_v[b][s]
            r2 = r * r
            r6 = r2 * r2 * r2
            r8 = r6 * r2
            rr2 = rr * rr
            rr6 = rr2 * rr2 * rr2
            rr8 = rr6 * rr2
            val_v[b][s] = -0.5 * (D3_S6 * c6 / (r6 + rr6)
                                  + D3_S8 * c8 / (r8 + rr8))
            iis_v[b][s] = ii_v[b][s]

        pltpu.async_copy(val_v[b], acc.at[iis_v[b]], sems[b], add=True)

        @pl.when(m < half - 1)
        def _():
            issue_lin(c + 2, b)

    wait_scat(0)
    wait_scat(1)
    _acc_to_out(acc, out_h)


def _add_body(x_ref, o_ref):
    o_ref[...] = x_ref[0] + x_ref[1]


_add_call = pl.pallas_call(
    _add_body,
    out_shape=jax.ShapeDtypeStruct((NP // 128, 128), jnp.float32),
)


def kernel(atomic_numbers, distances, idx_i, idx_j,
           d3_rcov, d3_rcn, d3_rc6, d3_r2r4):
    z = atomic_numbers.astype(jnp.int32)
    ii = idx_i.astype(jnp.int32)
    jj = idx_j.astype(jnp.int32)
    dist = distances.astype(jnp.float32)

    zp = jnp.pad(z, (0, NP - N_NODES))
    pad_e = EP - N_EDGES
    iip = jnp.pad(ii, (0, pad_e), constant_values=N_NODES)
    jjp = jnp.pad(jj, (0, pad_e))
    dp = jnp.pad(dist, (0, pad_e), constant_values=1.0)

    rcov96 = jnp.pad(d3_rcov.astype(jnp.float32), (0, 96 - N_ELEM))
    rcn480 = jnp.pad(d3_rcn.astype(jnp.float32).reshape(-1),
                     (0, 480 - N_ELEM * N_REF))
    q96 = jnp.pad(d3_r2r4.astype(jnp.float32), (0, 96 - N_ELEM))
    sq96 = jnp.sqrt(q96)
    rc6b = jnp.pad(
        d3_rc6.astype(jnp.bfloat16).reshape(N_ELEM * N_ELEM, N_REF * N_REF),
        ((0, 0), (0, 32 - N_REF * N_REF)))
    rc6p = lax.bitcast_convert_type(
        rc6b.reshape(N_ELEM * N_ELEM, 16, 2), jnp.int32)

    cn_parts, pidx = _cn_kernel(zp, iip, jjp, dp, rcov96)
    feat = _gw_kernel(cn_parts, zp, rcn480, q96, sq96)
    e_parts = _edisp_kernel(iip, jjp, dp, pidx, feat, rc6p)
    edisp = _add_call(e_parts.reshape(NC, NP // 128, 128))
    return edisp.reshape(NP)[:N_NODES]


# speedup vs baseline: 179.4300x; 1.1375x over previous
import functools

import jax
import jax.numpy as jnp
from jax import lax
from jax.experimental import pallas as pl
from jax.experimental.pallas import tpu as pltpu
from jax.experimental.pallas import tpu_sc as plsc

N_NODES = 100000
N_EDGES = 1600000
N_ELEM = 95
N_REF = 5

NC = 2
NS = 16
L = 16
NW = NC * NS

NB = 3136
NP = NB * NW
SL = NP // NS

EC = 1024
NCHUNK_A = (56, 42)
NCHUNK_C = (54, 44)
NTOT = 98
EP = EC * NS * NTOT

D3_K1 = 16.0
D3_K2 = 4.0 / 3.0
D3_K3 = -4.0
D3_S6 = 1.0
D3_S8 = 0.9171
D3_A1 = 0.3385
D3_A2 = 2.883
SQRT3 = 3.0 ** 0.5

_MESH = plsc.VectorSubcoreMesh(core_axis_name="c", subcore_axis_name="s",
                               num_cores=NC, num_subcores=NS)
_PARAMS = pltpu.CompilerParams(needs_layout_passes=False,
                               use_tc_tiling_on_sc=False)


def _edge_split(nchunk):
    n0, n1 = nchunk
    assert n0 % 2 == 0 and n1 % 2 == 0 and n0 + n1 == NTOT
    cid = lax.axis_index("c")
    sid = lax.axis_index("s")
    ebase = jnp.where(cid == 0, sid * (n0 * EC),
                      NS * (n0 * EC) + sid * (n1 * EC))
    half = jnp.where(cid == 0, n0 // 2, n1 // 2)
    return ebase, half


def _worker_id():
    return lax.axis_index("c") * NS + lax.axis_index("s")


def _zero_acc(zero_v, acc):
    sid = lax.axis_index("s")

    @pl.loop(0, SL // L)
    def _(i):
        zero_v[pl.ds(i * L, L)] = jnp.zeros((L,), jnp.float32)

    pltpu.sync_copy(zero_v, acc.at[pl.ds(sid * SL, SL)])
    plsc.subcore_barrier()


def _acc_to_out(acc, out_h):
    cid = lax.axis_index("c")
    sid = lax.axis_index("s")
    plsc.subcore_barrier()
    pltpu.sync_copy(acc.at[pl.ds(sid * SL, SL)],
                    out_h.at[pl.ds(cid * NP + sid * SL, SL)])


@functools.partial(
    pl.kernel,
    out_type=[jax.ShapeDtypeStruct((NC * NP,), jnp.float32),
              jax.ShapeDtypeStruct((EP,), jnp.int32)],
    mesh=_MESH,
    compiler_params=_PARAMS,
    scratch_types=[
        [pltpu.VMEM((EC,), jnp.int32)] * 2,
        [pltpu.VMEM((EC,), jnp.int32)] * 2,
        [pltpu.VMEM((EC,), jnp.float32)] * 2,
        [pltpu.VMEM((EC,), jnp.int32)] * 2,
        [pltpu.VMEM((EC,), jnp.int32)] * 2,
        [pltpu.VMEM((EC,), jnp.int32)] * 2,
        [pltpu.VMEM((EC,), jnp.float32)] * 2,
        [pltpu.VMEM((EC,), jnp.int32)] * 2,
        pltpu.VMEM((96,), jnp.float32),
        pltpu.VMEM((SL,), jnp.float32),
        pltpu.VMEM_SHARED((NP,), jnp.float32),
        [pltpu.SemaphoreType.DMA] * 2,
        [pltpu.SemaphoreType.DMA] * 2,
        [pltpu.SemaphoreType.DMA] * 2,
        [pltpu.SemaphoreType.DMA] * 2,
    ],
)
def _cn_kernel(z_h, ii_h, jj_h, d_h, rcov_h, cn_out, p_out,
               ii_v, jj_v, d_v, zi_v, zj_v, p_v, val_v, iis_v, rcov_v,
               zero_v, acc, seml, semg, semp, sems):
    pltpu.sync_copy(rcov_h, rcov_v)
    _zero_acc(zero_v, acc)

    ebase, half = _edge_split(NCHUNK_A)

    def base(c):
        return ebase + c * EC

    def issue_lin(c, b):
        pltpu.async_copy(ii_h.at[pl.ds(base(c), EC)], ii_v[b], seml[b])
        pltpu.async_copy(jj_h.at[pl.ds(base(c), EC)], jj_v[b], seml[b])
        pltpu.async_copy(d_h.at[pl.ds(base(c), EC)], d_v[b], seml[b])

    def wait_lin(b):
        pltpu.make_async_copy(ii_h.at[pl.ds(0, EC)], ii_v[b], seml[b]).wait()
        pltpu.make_async_copy(jj_h.at[pl.ds(0, EC)], jj_v[b], seml[b]).wait()
        pltpu.make_async_copy(d_h.at[pl.ds(0, EC)], d_v[b], seml[b]).wait()

    def issue_gat(b):
        pltpu.async_copy(z_h.at[ii_v[b]], zi_v[b], semg[b])
        pltpu.async_copy(z_h.at[jj_v[b]], zj_v[b], semg[b])

    def wait_gat(b):
        pltpu.make_async_copy(z_h.at[pl.ds(0, EC)], zi_v[b], semg[b]).wait()
        pltpu.make_async_copy(z_h.at[pl.ds(0, EC)], zj_v[b], semg[b]).wait()

    def wait_pwrite(b):
        pltpu.make_async_copy(p_v[b], p_out.at[pl.ds(0, EC)], semp[b]).wait()

    def wait_scat(b):
        pltpu.make_async_copy(val_v[b], acc.at[iis_v[b]], sems[b]).wait()

    issue_lin(0, 0)
    wait_lin(0)
    issue_gat(0)
    issue_lin(1, 1)

    @pl.loop(0, half)
    def _pair(m):
        for b in (0, 1):
            c = 2 * m + b
            o = 1 - b
            if b == 0:
                wait_lin(o)
                issue_gat(o)
            else:
                @pl.when(m < half - 1)
                def _():
                    wait_lin(o)
                    issue_gat(o)
            wait_gat(b)

            @pl.when(m >= 1)
            def _():
                wait_pwrite(b)
                wait_scat(b)

            @pl.loop(0, EC // L)
            def _(i):
                s = pl.ds(i * L, L)
                zi = zi_v[b][s]
                zj = zj_v[b][s]
                ri = plsc.load_gather(rcov_v, [zi])
                rj = plsc.load_gather(rcov_v, [zj])
                rco = D3_K2 * (ri + rj)
                t = jnp.exp(-D3_K1 * (rco / d_v[b][s] - 1.0))
                val_v[b][s] = 1.0 / (1.0 + t)
                p_v[b][s] = zi * N_ELEM + zj
                iis_v[b][s] = ii_v[b][s]

            pltpu.async_copy(p_v[b], p_out.at[pl.ds(base(c), EC)], semp[b])
            pltpu.async_copy(val_v[b], acc.at[iis_v[b]], sems[b], add=True)

            @pl.when(m < half - 1)
            def _():
                issue_lin(c + 2, b)

    wait_pwrite(0)
    wait_pwrite(1)
    wait_scat(0)
    wait_scat(1)
    _acc_to_out(acc, cn_out)


@functools.partial(
    pl.kernel,
    out_type=jax.ShapeDtypeStruct((NP, 8), jnp.float32),
    mesh=_MESH,
    compiler_params=_PARAMS,
    scratch_types=[
        pltpu.VMEM((NB,), jnp.float32),
        pltpu.VMEM((NB,), jnp.float32),
        pltpu.VMEM((NB,), jnp.int32),
        pltpu.VMEM((NB, 8), jnp.float32),
        pltpu.VMEM((480,), jnp.float32),
        pltpu.VMEM((96,), jnp.float32),
        pltpu.VMEM((96,), jnp.float32),
    ],
)
def _gw_kernel(cn_h, z_h, rcn_h, q_h, sq_h, out_h,
               cn0_v, cn1_v, z_v, feat_v, rcn_v, q_v, sq_v):
    wid = _worker_id()
    nb = wid * NB
    pltpu.sync_copy(rcn_h, rcn_v)
    pltpu.sync_copy(q_h, q_v)
    pltpu.sync_copy(sq_h, sq_v)
    pltpu.sync_copy(cn_h.at[pl.ds(nb, NB)], cn0_v)
    pltpu.sync_copy(cn_h.at[pl.ds(NP + nb, NB)], cn1_v)
    pltpu.sync_copy(z_h.at[pl.ds(nb, NB)], z_v)

    iota = lax.iota(jnp.int32, L)
    cols = [jnp.full((L,), k, jnp.int32) for k in range(8)]

    @pl.loop(0, NB // L)
    def _(i):
        s = pl.ds(i * L, L)
        z = z_v[s]
        cn = cn0_v[s] + cn1_v[s]
        zb = z * N_REF
        r = [plsc.load_gather(rcn_v, [zb + k]) for k in range(N_REF)]
        maxcn = r[0]
        for k in range(1, N_REF):
            maxcn = jnp.maximum(maxcn, r[k])
        w = []
        norm = None
        for k in range(N_REF):
            d = cn - r[k]
            wk = jnp.exp(D3_K3 * d * d)
            w.append(wk)
            norm = wk if norm is None else norm + wk
        exc = norm < 1e-30
        safe = jnp.where(exc, 1.0, norm)
        rows = i * L + iota
        for k in range(N_REF):
            gwk = jnp.where(exc, jnp.where(r[k] == maxcn, 1.0, 0.0),
                            w[k] / safe)
            plsc.store_scatter(feat_v, [rows, cols[k]], gwk)
        plsc.store_scatter(feat_v, [rows, cols[5]],
                           plsc.load_gather(q_v, [z]))
        plsc.store_scatter(feat_v, [rows, cols[6]],
                           plsc.load_gather(sq_v, [z]))
        plsc.store_scatter(feat_v, [rows, cols[7]], z.astype(jnp.float32))

    pltpu.sync_copy(feat_v, out_h.at[pl.ds(nb, NB), :])


@functools.partial(
    pl.kernel,
    out_type=jax.ShapeDtypeStruct((NC * NP,), jnp.float32),
    mesh=_MESH,
    compiler_params=_PARAMS,
    scratch_types=[
        [pltpu.VMEM((EC,), jnp.int32)] * 2,
        [pltpu.VMEM((EC,), jnp.int32)] * 2,
        [pltpu.VMEM((EC,), jnp.int32)] * 2,
        [pltpu.VMEM((EC,), jnp.float32)] * 2,
        [pltpu.VMEM((EC, 8), jnp.float32)] * 2,
        [pltpu.VMEM((EC, 8), jnp.float32)] * 2,
        [pltpu.VMEM((EC, 16), jnp.int32)] * 2,
        [pltpu.VMEM((EC,), jnp.float32)] * 2,
        [pltpu.VMEM((EC,), jnp.int32)] * 2,
        pltpu.VMEM((SL,), jnp.float32),
        pltpu.VMEM_SHARED((NP,), jnp.float32),
        [pltpu.SemaphoreType.DMA] * 2,
        [pltpu.SemaphoreType.DMA] * 2,
        [pltpu.SemaphoreType.DMA] * 2,
    ],
)
def _edisp_kernel(ii_h, jj_h, d_h, p_h, feat_h, rc6_h, out_h,
                  ii_v, jj_v, p_v, d_v, wi_v, wj_v, rows_v, val_v, iis_v,
                  zero_v, acc, seml, semg, sems):
    _zero_acc(zero_v, acc)

    ebase, half = _edge_split(NCHUNK_C)
    iota = lax.iota(jnp.int32, L)
    cols = [jnp.full((L,), k, jnp.int32) for k in range(16)]

    def base(c):
        return ebase + c * EC

    def issue_lin(c, b):
        pltpu.async_copy(ii_h.at[pl.ds(base(c), EC)], ii_v[b], seml[b])
        pltpu.async_copy(jj_h.at[pl.ds(base(c), EC)], jj_v[b], seml[b])
        pltpu.async_copy(p_h.at[pl.ds(base(c), EC)], p_v[b], seml[b])
        pltpu.async_copy(d_h.at[pl.ds(base(c), EC)], d_v[b], seml[b])

    def wait_lin(b):
        pltpu.make_async_copy(ii_h.at[pl.ds(0, EC)], ii_v[b], seml[b]).wait()
        pltpu.make_async_copy(jj_h.at[pl.ds(0, EC)], jj_v[b], seml[b]).wait()
        pltpu.make_async_copy(p_h.at[pl.ds(0, EC)], p_v[b], seml[b]).wait()
        pltpu.make_async_copy(d_h.at[pl.ds(0, EC)], d_v[b], seml[b]).wait()

    def issue_gat(b):
        pltpu.async_copy(feat_h.at[ii_v[b]], wi_v[b], semg[b])
        pltpu.async_copy(feat_h.at[jj_v[b]], wj_v[b], semg[b])
        pltpu.async_copy(rc6_h.at[p_v[b]], rows_v[b], semg[b])

    def wait_gat(b):
        pltpu.make_async_copy(feat_h.at[pl.ds(0, EC), :], wi_v[b],
                              semg[b]).wait()
        pltpu.make_async_copy(feat_h.at[pl.ds(0, EC), :], wj_v[b],
                              semg[b]).wait()
        pltpu.make_async_copy(rc6_h.at[pl.ds(0, EC), :], rows_v[b],
                              semg[b]).wait()

    def wait_scat(b):
        pltpu.make_async_copy(val_v[b], acc.at[iis_v[b]], sems[b]).wait()

    issue_lin(0, 0)
    wait_lin(0)
    issue_gat(0)
    issue_lin(1, 1)

    @pl.loop(0, half)
    def _pair(m):
      for b in (0, 1):
        c = 2 * m + b
        o = 1 - b
        if b == 0:
            wait_lin(o)
            issue_gat(o)
        else:
            @pl.when(m < half - 1)
            def _():
                wait_lin(o)
                issue_gat(o)
        wait_gat(b)

        @pl.when(m >= 1)
        def _():
            wait_scat(b)

        @pl.loop(0, EC // L)
        def _(i):
            s = pl.ds(i * L, L)
            rows = i * L + iota
            wi = [plsc.load_gather(wi_v[b], [rows, cols[a]])
                  for a in range(N_REF)]
            wj = [plsc.load_gather(wj_v[b], [rows, cols[k]])
                  for k in range(N_REF)]
            c6 = None
            for w in range(13):
                word = plsc.load_gather(rows_v[b], [rows, cols[w]])
                bfp = plsc.bitcast(word, jnp.bfloat16)
                ev, od = plsc.unpack(bfp, format=plsc.PackFormat.INTERLEAVED)
                a0, b0 = divmod(2 * w, 5)
                t = (wi[a0] * wj[b0]) * ev
                c6 = t if c6 is None else c6 + t
                if 2 * w + 1 < N_REF * N_REF:
                    a1, b1 = divmod(2 * w + 1, 5)
                    c6 = c6 + (wi[a1] * wj[b1]) * od
            qi = plsc.load_gather(wi_v[b], [rows, cols[5]])
            qj = plsc.load_gather(wj_v[b], [rows, cols[5]])
            sqi = plsc.load_gather(wi_v[b], [rows, cols[6]])
            sqj = plsc.load_gather(wj_v[b], [rows, cols[6]])
            qq = 3.0 * qi * qj
            c8 = c6 * qq
            rr = D3_A1 * SQRT3 * sqi * sqj + D3_A2
            r = d_v[b][s]
            r2 = r * r
            r6 = r2 * r2 * r2
            r8 = r6 * r2
            rr2 = rr * rr
            rr6 = rr2 * rr2 * rr2
            rr8 = rr6 * rr2
            val_v[b][s] = -0.5 * (D3_S6 * c6 / (r6 + rr6)
                                  + D3_S8 * c8 / (r8 + rr8))
            iis_v[b][s] = ii_v[b][s]

        pltpu.async_copy(val_v[b], acc.at[iis_v[b]], sems[b], add=True)

        @pl.when(m < half - 1)
        def _():
            issue_lin(c + 2, b)

    wait_scat(0)
    wait_scat(1)
    _acc_to_out(acc, out_h)


def _add_body(x_ref, o_ref):
    o_ref[...] = x_ref[0] + x_ref[1]


_add_call = pl.pallas_call(
    _add_body,
    out_shape=jax.ShapeDtypeStruct((NP // 128, 128), jnp.float32),
)


def kernel(atomic_numbers, distances, idx_i, idx_j,
           d3_rcov, d3_rcn, d3_rc6, d3_r2r4):
    z = atomic_numbers.astype(jnp.int32)
    ii = idx_i.astype(jnp.int32)
    jj = idx_j.astype(jnp.int32)
    dist = distances.astype(jnp.float32)

    zp = jnp.pad(z, (0, NP - N_NODES))
    pad_e = EP - N_EDGES
    iip = jnp.pad(ii, (0, pad_e), constant_values=N_NODES)
    jjp = jnp.pad(jj, (0, pad_e))
    dp = jnp.pad(dist, (0, pad_e), constant_values=1.0)

    rcov96 = jnp.pad(d3_rcov.astype(jnp.float32), (0, 96 - N_ELEM))
    rcn480 = jnp.pad(d3_rcn.astype(jnp.float32).reshape(-1),
                     (0, 480 - N_ELEM * N_REF))
    q96 = jnp.pad(d3_r2r4.astype(jnp.float32), (0, 96 - N_ELEM))
    sq96 = jnp.sqrt(q96)
    rc6b = jnp.pad(
        d3_rc6.astype(jnp.bfloat16).reshape(N_ELEM * N_ELEM, N_REF * N_REF),
        ((0, 0), (0, 32 - N_REF * N_REF)))
    rc6p = lax.bitcast_convert_type(
        rc6b.reshape(N_ELEM * N_ELEM, 16, 2), jnp.int32)

    cn_parts, pidx = _cn_kernel(zp, iip, jjp, dp, rcov96)
    feat = _gw_kernel(cn_parts, zp, rcn480, q96, sq96)
    e_parts = _edisp_kernel(iip, jjp, dp, pidx, feat, rc6p)
    edisp = _add_call(e_parts.reshape(NC, NP // 128, 128))
    return edisp.reshape(NP)[:N_NODES]
